# Initial kernel scaffold; baseline (speedup 1.0000x reference)
import jax
import jax.numpy as jnp
from jax.experimental import pallas as pl


def kernel(hidden_states, gate_weight):
    raise NotImplementedError("write your pallas kernel here")


# speedup vs baseline: 1.3603x; 1.3603x over previous
import functools

import jax
import jax.numpy as jnp
from jax.experimental import pallas as pl

_HIDDEN = 1024
_EXPERTS = 64
_TOKENS = 32768
_BLK = 512


def _router_block(h_ref, w_ref, weights_ref, idx_ref, logits_ref):
    logits = jnp.dot(h_ref[...], w_ref[...], preferred_element_type=jnp.float32)
    logits_ref[...] = logits

    ids = jax.lax.broadcasted_iota(jnp.int32, logits.shape, 1)
    m1 = jnp.max(logits, axis=1, keepdims=True)
    i1 = jnp.min(jnp.where(logits == m1, ids, _EXPERTS), axis=1, keepdims=True)
    masked = jnp.where(ids == i1, -jnp.inf, logits)
    m2 = jnp.max(masked, axis=1, keepdims=True)
    i2 = jnp.min(jnp.where(masked == m2, ids, _EXPERTS), axis=1, keepdims=True)

    e = jnp.exp(m2 - m1)
    w1 = 1.0 / (1.0 + e)
    weights_ref[...] = jnp.concatenate([w1, 1.0 - w1], axis=1)
    idx_ref[...] = jnp.concatenate([i1, i2], axis=1)


@functools.partial(jax.jit, static_argnames=())
def kernel(hidden_states, gate_weight):
    wt = gate_weight.T
    grid = (_TOKENS // _BLK,)
    out = pl.pallas_call(
        _router_block,
        grid=grid,
        in_specs=[
            pl.BlockSpec((_BLK, _HIDDEN), lambda i: (i, 0)),
            pl.BlockSpec((_HIDDEN, _EXPERTS), lambda i: (0, 0)),
        ],
        out_specs=[
            pl.BlockSpec((_BLK, 2), lambda i: (i, 0)),
            pl.BlockSpec((_BLK, 2), lambda i: (i, 0)),
            pl.BlockSpec((_BLK, _EXPERTS), lambda i: (i, 0)),
        ],
        out_shape=[
            jax.ShapeDtypeStruct((_TOKENS, 2), jnp.float32),
            jax.ShapeDtypeStruct((_TOKENS, 2), jnp.int32),
            jax.ShapeDtypeStruct((_TOKENS, _EXPERTS), jnp.float32),
        ],
    )(hidden_states, wt)
    return (out[0], out[1], out[2])


# speedup vs baseline: 1.7948x; 1.3194x over previous
import functools

import jax
import jax.numpy as jnp
from jax.experimental import pallas as pl
from jax.experimental.pallas import tpu as pltpu

_HIDDEN = 1024
_EXPERTS = 64
_TOKENS = 32768
_BLK = 2048


def _router_block(h_ref, w_ref, weights_ref, idx_ref, logits_ref):
    logits = jnp.dot(h_ref[...], w_ref[...], preferred_element_type=jnp.float32)
    logits_ref[...] = logits

    ids = jax.lax.broadcasted_iota(jnp.int32, logits.shape, 1)
    m1 = jnp.max(logits, axis=1, keepdims=True)
    i1 = jnp.min(jnp.where(logits == m1, ids, _EXPERTS), axis=1, keepdims=True)
    masked = jnp.where(ids == i1, -jnp.inf, logits)
    m2 = jnp.max(masked, axis=1, keepdims=True)
    i2 = jnp.min(jnp.where(masked == m2, ids, _EXPERTS), axis=1, keepdims=True)

    e = jnp.exp(m2 - m1)
    w1 = 1.0 / (1.0 + e)
    weights_ref[...] = jnp.concatenate([w1, 1.0 - w1], axis=1)
    idx_ref[...] = jnp.concatenate([i1, i2], axis=1)


@functools.partial(jax.jit, static_argnames=())
def kernel(hidden_states, gate_weight):
    wt = gate_weight.T
    grid = (_TOKENS // _BLK,)
    out = pl.pallas_call(
        _router_block,
        grid=grid,
        in_specs=[
            pl.BlockSpec((_BLK, _HIDDEN), lambda i: (i, 0)),
            pl.BlockSpec((_HIDDEN, _EXPERTS), lambda i: (0, 0)),
        ],
        out_specs=[
            pl.BlockSpec((_BLK, 2), lambda i: (i, 0)),
            pl.BlockSpec((_BLK, 2), lambda i: (i, 0)),
            pl.BlockSpec((_BLK, _EXPERTS), lambda i: (i, 0)),
        ],
        out_shape=[
            jax.ShapeDtypeStruct((_TOKENS, 2), jnp.float32),
            jax.ShapeDtypeStruct((_TOKENS, 2), jnp.int32),
            jax.ShapeDtypeStruct((_TOKENS, _EXPERTS), jnp.float32),
        ],
        compiler_params=pltpu.CompilerParams(
            dimension_semantics=("parallel",),
        ),
    )(hidden_states, wt)
    return (out[0], out[1], out[2])


# speedup vs baseline: 1.8861x; 1.0509x over previous
import functools

import jax
import jax.numpy as jnp
from jax.experimental import pallas as pl
from jax.experimental.pallas import tpu as pltpu

_HIDDEN = 1024
_EXPERTS = 64
_TOKENS = 32768
_BLK = 4096


def _router_block(h_ref, w_ref, weights_ref, idx_ref, logits_ref):
    logits = jnp.dot(h_ref[...], w_ref[...], preferred_element_type=jnp.float32)
    logits_ref[...] = logits

    ids = jax.lax.broadcasted_iota(jnp.int32, logits.shape, 1)
    m1 = jnp.max(logits, axis=1, keepdims=True)
    i1 = jnp.min(jnp.where(logits == m1, ids, _EXPERTS), axis=1, keepdims=True)
    masked = jnp.where(ids == i1, -jnp.inf, logits)
    m2 = jnp.max(masked, axis=1, keepdims=True)
    i2 = jnp.min(jnp.where(masked == m2, ids, _EXPERTS), axis=1, keepdims=True)

    e = jnp.exp(m2 - m1)
    w1 = 1.0 / (1.0 + e)
    weights_ref[...] = jnp.concatenate([w1, 1.0 - w1], axis=1)
    idx_ref[...] = jnp.concatenate([i1, i2], axis=1)


@functools.partial(jax.jit, static_argnames=())
def kernel(hidden_states, gate_weight):
    wt = gate_weight.T
    grid = (_TOKENS // _BLK,)
    out = pl.pallas_call(
        _router_block,
        grid=grid,
        in_specs=[
            pl.BlockSpec((_BLK, _HIDDEN), lambda i: (i, 0)),
            pl.BlockSpec((_HIDDEN, _EXPERTS), lambda i: (0, 0)),
        ],
        out_specs=[
            pl.BlockSpec((_BLK, 2), lambda i: (i, 0)),
            pl.BlockSpec((_BLK, 2), lambda i: (i, 0)),
            pl.BlockSpec((_BLK, _EXPERTS), lambda i: (i, 0)),
        ],
        out_shape=[
            jax.ShapeDtypeStruct((_TOKENS, 2), jnp.float32),
            jax.ShapeDtypeStruct((_TOKENS, 2), jnp.int32),
            jax.ShapeDtypeStruct((_TOKENS, _EXPERTS), jnp.float32),
        ],
        compiler_params=pltpu.CompilerParams(
            dimension_semantics=("parallel",),
        ),
    )(hidden_states, wt)
    return (out[0], out[1], out[2])
